# Initial kernel scaffold; baseline (speedup 1.0000x reference)
import jax
import jax.numpy as jnp
from jax.experimental import pallas as pl


def kernel(x, edge_index, W1, b1, W2, b2, W3, b3, L1W, L1b, L2W, L2b, L3W, L3b):
    raise NotImplementedError("write your pallas kernel here")


# speedup vs baseline: 16.2596x; 16.2596x over previous
import functools

import jax
import jax.numpy as jnp
from jax import lax
from jax.experimental import pallas as pl
from jax.experimental.pallas import tpu as pltpu
from jax.experimental.pallas import tpu_sc as plsc

N = 10000
E = 320000
D = 128
NC = 2
NS = 16
NW = NC * NS
SLAB_A = 624
SLAB_B = N - (NS - 1) * SLAB_A
CH = 80
NSC = 25
STG_E = E // NW // (NSC * CH)
STG_F = E // NS // (NSC * CH)
NCH_E = E // NW // CH
EPW = E // NW

_MESH = plsc.VectorSubcoreMesh(core_axis_name="c", subcore_axis_name="s")
_F32 = jnp.float32


def _slabbed(sid, emit):
    @pl.when(sid < NS - 1)
    def _():
        emit(pl.multiple_of(sid * SLAB_A, 8), SLAB_A)

    @pl.when(sid == NS - 1)
    def _():
        emit((NS - 1) * SLAB_A, SLAB_B)


def _make_deg():
    @functools.partial(
        pl.kernel,
        out_type=[jax.ShapeDtypeStruct((N, D), _F32),
                  jax.ShapeDtypeStruct((N, D), _F32)],
        mesh=_MESH,
        scratch_types=[
            pltpu.VMEM((NCH_E, CH), jnp.int32),
            pltpu.VMEM((CH, D), _F32),
            pltpu.VMEM((CH, D), _F32),
            pltpu.VMEM_SHARED((N, D), _F32),
        ],
    )
    def degk(dstd, out0, out1, dst_v, ones_v, zeros_v, acc):
        cid = lax.axis_index("c")
        sid = lax.axis_index("s")
        w = cid * NS + sid
        pltpu.sync_copy(dstd.at[w], dst_v)
        _fill2d(ones_v, CH, 1.0)
        _fill2d(zeros_v, CH, 0.0)
        _slabbed(sid, lambda r0, n: _zero_slab(zeros_v, acc, r0, n))
        plsc.subcore_barrier()

        def body(i, _):
            pltpu.sync_copy(ones_v, acc.at[dst_v.at[i]], add=True)
            return 0
        lax.fori_loop(0, NCH_E, body, 0)
        plsc.subcore_barrier()

        @pl.when(cid == 0)
        def _():
            _slabbed(sid, lambda r0, n: pltpu.sync_copy(
                acc.at[pl.ds(r0, n)], out0.at[pl.ds(r0, n)]))

        @pl.when(cid == 1)
        def _():
            _slabbed(sid, lambda r0, n: pltpu.sync_copy(
                acc.at[pl.ds(r0, n)], out1.at[pl.ds(r0, n)]))

    return degk


def _fill2d(ref, nrows, val):
    def body(i, _):
        r = i // 8
        c = lax.rem(i, 8) * 16
        ref[r, pl.ds(c, 16)] = jnp.full((16,), val, _F32)
        return 0
    lax.fori_loop(0, nrows * 8, body, 0)


def _zero_slab(zeros_v, acc, r0, n):
    for off in range(0, n, CH):
        m = min(CH, n - off)
        pltpu.sync_copy(zeros_v.at[pl.ds(0, m)], acc.at[pl.ds(r0 + off, m)])


def _edge_phase(table, acc, src3, dst3, w, nstages,
                src_v, dst_v, rows0, rows1, sem0, sem1):
    for st in range(nstages):
        pltpu.sync_copy(src3.at[w * nstages + st], src_v)
        pltpu.sync_copy(dst3.at[w * nstages + st], dst_v)
        _edge_pipeline(table, acc, src_v, dst_v, rows0, rows1,
                       sem0, sem1, NSC)


def _edge_pipeline(table, acc, src_v, dst_v, rows0, rows1, sem0, sem1, nch):
    def pair(i):
        d0 = pltpu.async_copy(table.at[src_v.at[i]], rows0, sem0)
        d1 = pltpu.async_copy(table.at[src_v.at[i + 1]], rows1, sem1)
        d0.wait()
        pltpu.sync_copy(rows0, acc.at[dst_v.at[i]], add=True)
        d1.wait()
        pltpu.sync_copy(rows1, acc.at[dst_v.at[i + 1]], add=True)

    def body(p, _):
        pair(2 * p)
        return 0
    lax.fori_loop(0, nch // 2, body, 0)
    if nch % 2:
        i = nch - 1
        pltpu.async_copy(table.at[src_v.at[i]], rows0, sem0).wait()
        pltpu.sync_copy(rows0, acc.at[dst_v.at[i]], add=True)


def _make_spmm_e():
    @functools.partial(
        pl.kernel,
        out_type=[jax.ShapeDtypeStruct((N, D), _F32),
                  jax.ShapeDtypeStruct((N, D), _F32)],
        mesh=_MESH,
        scratch_types=[
            pltpu.VMEM((NSC, CH), jnp.int32),
            pltpu.VMEM((NSC, CH), jnp.int32),
            pltpu.VMEM((CH, D), _F32),
            pltpu.VMEM((CH, D), _F32),
            pltpu.VMEM_SHARED((N, D), _F32),
            pltpu.SemaphoreType.DMA,
            pltpu.SemaphoreType.DMA,
        ],
    )
    def spmm_e(xs, src3, dst3, zhbm, out0, out1,
               src_v, dst_v, rows0, rows1, acc, sem0, sem1):
        cid = lax.axis_index("c")
        sid = lax.axis_index("s")
        w = cid * NS + sid

        @pl.when(cid == 0)
        def _():
            _slabbed(sid, lambda r0, n: pltpu.sync_copy(
                xs.at[pl.ds(r0, n)], acc.at[pl.ds(r0, n)]))

        @pl.when(cid == 1)
        def _():
            _slabbed(sid, lambda r0, n: pltpu.sync_copy(
                zhbm.at[pl.ds(0, n)], acc.at[pl.ds(r0, n)]))

        plsc.subcore_barrier()
        _edge_phase(xs, acc, src3, dst3, w, STG_E,
                    src_v, dst_v, rows0, rows1, sem0, sem1)
        plsc.subcore_barrier()

        @pl.when(cid == 0)
        def _():
            _slabbed(sid, lambda r0, n: pltpu.sync_copy(
                acc.at[pl.ds(r0, n)], out0.at[pl.ds(r0, n)]))

        @pl.when(cid == 1)
        def _():
            _slabbed(sid, lambda r0, n: pltpu.sync_copy(
                acc.at[pl.ds(r0, n)], out1.at[pl.ds(r0, n)]))

    return spmm_e


def _make_spmm_f():
    @functools.partial(
        pl.kernel,
        out_type=[jax.ShapeDtypeStruct((N, D), _F32),
                  jax.ShapeDtypeStruct((N, D), _F32)],
        mesh=_MESH,
        scratch_types=[
            pltpu.VMEM((NSC, CH), jnp.int32),
            pltpu.VMEM((NSC, CH), jnp.int32),
            pltpu.VMEM((CH, D), _F32),
            pltpu.VMEM((CH, D), _F32),
            pltpu.VMEM_SHARED((N, D), _F32),
            pltpu.SemaphoreType.DMA,
            pltpu.SemaphoreType.DMA,
        ],
    )
    def spmm_f(xs0, xs1, src3, dst3, out0, out1,
               src_v, dst_v, rows0, rows1, acc, sem0, sem1):
        cid = lax.axis_index("c")
        sid = lax.axis_index("s")

        @pl.when(cid == 0)
        def _():
            _slabbed(sid, lambda r0, n: pltpu.sync_copy(
                xs0.at[pl.ds(r0, n)], acc.at[pl.ds(r0, n)]))

        @pl.when(cid == 1)
        def _():
            _slabbed(sid, lambda r0, n: pltpu.sync_copy(
                xs1.at[pl.ds(r0, n)], acc.at[pl.ds(r0, n)]))

        plsc.subcore_barrier()

        @pl.when(cid == 0)
        def _():
            _edge_phase(xs0, acc, src3, dst3, sid, STG_F,
                        src_v, dst_v, rows0, rows1, sem0, sem1)

        @pl.when(cid == 1)
        def _():
            _edge_phase(xs1, acc, src3, dst3, sid, STG_F,
                        src_v, dst_v, rows0, rows1, sem0, sem1)

        plsc.subcore_barrier()

        @pl.when(cid == 0)
        def _():
            _slabbed(sid, lambda r0, n: pltpu.sync_copy(
                acc.at[pl.ds(r0, n)], out0.at[pl.ds(r0, n)]))

        @pl.when(cid == 1)
        def _():
            _slabbed(sid, lambda r0, n: pltpu.sync_copy(
                acc.at[pl.ds(r0, n)], out1.at[pl.ds(r0, n)]))

    return spmm_f


_DEG = _make_deg()
_SPMM_E = _make_spmm_e()
_SPMM_F = _make_spmm_f()

_BM = 1000


def _dinv_col(h0, h1):
    def body(h0_ref, h1_ref, o):
        deg = h0_ref[:, :1] + h1_ref[:, :1] + 1.0
        o[...] = lax.rsqrt(deg)

    return pl.pallas_call(
        body,
        grid=(N // _BM,),
        in_specs=[pl.BlockSpec((_BM, D), lambda m: (m, 0)),
                  pl.BlockSpec((_BM, D), lambda m: (m, 0))],
        out_specs=pl.BlockSpec((_BM, 1), lambda m: (m, 0)),
        out_shape=jax.ShapeDtypeStruct((N, 1), _F32),
    )(h0, h1)


def _scale(x, dinv):
    def body(x_ref, dv, oxs):
        oxs[...] = x_ref[...] * dv[...]

    return pl.pallas_call(
        body,
        grid=(N // _BM,),
        in_specs=[pl.BlockSpec((_BM, D), lambda m: (m, 0)),
                  pl.BlockSpec((_BM, 1), lambda m: (m, 0))],
        out_specs=pl.BlockSpec((_BM, D), lambda m: (m, 0)),
        out_shape=jax.ShapeDtypeStruct((N, D), _F32),
    )(x, dinv)


def _layer1(a0, a1, dinv, w1t, b1r):
    def body(a0_ref, a1_ref, dv, w, b, o0, o1):
        di = dv[...]
        agg = (a0_ref[...] + a1_ref[...]) * di
        h = jnp.dot(agg, w[...], preferred_element_type=_F32) + b[...]
        xs2 = jnp.maximum(h, 0.0) * di
        o0[...] = xs2[:, :D]
        o1[...] = xs2[:, D:]

    return pl.pallas_call(
        body,
        grid=(N // _BM,),
        in_specs=[pl.BlockSpec((_BM, D), lambda m: (m, 0)),
                  pl.BlockSpec((_BM, D), lambda m: (m, 0)),
                  pl.BlockSpec((_BM, 1), lambda m: (m, 0)),
                  pl.BlockSpec((D, 256), lambda m: (0, 0)),
                  pl.BlockSpec((1, 256), lambda m: (0, 0))],
        out_specs=[pl.BlockSpec((_BM, D), lambda m: (m, 0)),
                   pl.BlockSpec((_BM, D), lambda m: (m, 0))],
        out_shape=[jax.ShapeDtypeStruct((N, D), _F32),
                   jax.ShapeDtypeStruct((N, D), _F32)],
    )(a0, a1, dinv, w1t, b1r)


def _layer2(g0, g1, dinv, w2t, b2r, w3t):
    def body(g0_ref, g1_ref, dv, w2, b2, w3, o):
        di = dv[...]
        agg = jnp.concatenate([g0_ref[...], g1_ref[...]], axis=1) * di
        h2 = jnp.maximum(
            jnp.dot(agg, w2[...], preferred_element_type=_F32) + b2[...], 0.0)
        y = jnp.dot(h2, w3[...], preferred_element_type=_F32)
        o[...] = y * di

    return pl.pallas_call(
        body,
        grid=(N // _BM,),
        in_specs=[pl.BlockSpec((_BM, D), lambda m: (m, 0)),
                  pl.BlockSpec((_BM, D), lambda m: (m, 0)),
                  pl.BlockSpec((_BM, 1), lambda m: (m, 0)),
                  pl.BlockSpec((256, 256), lambda m: (0, 0)),
                  pl.BlockSpec((1, 256), lambda m: (0, 0)),
                  pl.BlockSpec((256, D), lambda m: (0, 0))],
        out_specs=pl.BlockSpec((_BM, D), lambda m: (m, 0)),
        out_shape=jax.ShapeDtypeStruct((N, D), _F32),
    )(g0, g1, dinv, w2t, b2r, w3t)


def _layer3(c0, c1, dinv, b3r):
    def body(c0_ref, c1_ref, dv, b, o):
        di = dv[...]
        agg = (c0_ref[...] + c1_ref[...]) * di
        o[...] = jnp.maximum(agg + b[...], 0.0)

    return pl.pallas_call(
        body,
        grid=(N // _BM,),
        in_specs=[pl.BlockSpec((_BM, D), lambda m: (m, 0)),
                  pl.BlockSpec((_BM, D), lambda m: (m, 0)),
                  pl.BlockSpec((_BM, 1), lambda m: (m, 0)),
                  pl.BlockSpec((1, D), lambda m: (0, 0))],
        out_specs=pl.BlockSpec((_BM, D), lambda m: (m, 0)),
        out_shape=jax.ShapeDtypeStruct((N, D), _F32),
    )(c0, c1, dinv, b3r)


def _mlp(hr, w1t, b1r, w2t, b2r, w3row, b3s):
    def body(h, w1, b1, w2, b2, w3, b3, o):
        z = jnp.dot(h[...], w1[...], preferred_element_type=_F32) + b1[...]
        z = jnp.where(z > 0, z, 0.1 * z)
        z = jnp.dot(z, w2[...], preferred_element_type=_F32) + b2[...]
        z = jnp.where(z > 0, z, 0.1 * z)
        t = jnp.sum(z * w3[...], axis=1, keepdims=True) + b3[...]
        o[...] = 1.0 / (1.0 + jnp.exp(-t))

    m = hr.shape[0]
    return pl.pallas_call(
        body,
        out_shape=jax.ShapeDtypeStruct((m, 1), _F32),
    )(hr, w1t, b1r, w2t, b2r, w3row, b3s)


def kernel(x, edge_index, W1, b1, W2, b2, W3, b3,
           L1W, L1b, L2W, L2b, L3W, L3b):
    src = edge_index[0]
    dst = edge_index[1]
    src3 = src.reshape(E // (NSC * CH), NSC, CH)
    dst3 = dst.reshape(E // (NSC * CH), NSC, CH)
    dstd = dst.reshape(NW, NCH_E, CH)
    zhbm = jnp.zeros((SLAB_B, D), _F32)

    h0, h1 = _DEG(dstd)
    dinv = _dinv_col(h0, h1)
    xs = _scale(x, dinv)
    a0, a1 = _SPMM_E(xs, src3, dst3, zhbm)
    x2a, x2b = _layer1(a0, a1, dinv, W1.T, b1.reshape(1, -1))
    g0, g1 = _SPMM_F(x2a, x2b, src3, dst3)
    xs3 = _layer2(g0, g1, dinv, W2.T, b2.reshape(1, -1), W3.T)
    c0, c1 = _SPMM_E(xs3, src3, dst3, zhbm)
    h3 = _layer3(c0, c1, dinv, b3.reshape(1, -1))
    hr = h3.reshape(N // 40, 40 * D)
    return _mlp(hr, L1W.T, L1b.reshape(1, -1), L2W.T, L2b.reshape(1, -1),
                L3W.reshape(1, -1), L3b.reshape(1, 1))


# speedup vs baseline: 22.7760x; 1.4008x over previous
import functools

import jax
import jax.numpy as jnp
from jax import lax
from jax.experimental import pallas as pl
from jax.experimental.pallas import tpu as pltpu
from jax.experimental.pallas import tpu_sc as plsc

N = 10000
E = 320000
D = 128
NC = 2
NS = 16
NW = NC * NS
SLAB_A = 624
SLAB_B = N - (NS - 1) * SLAB_A
CH = 80
NSC = 25
STG_E = E // NW // (NSC * CH)
STG_F = E // NS // (NSC * CH)
NCH_E = E // NW // CH
EPW = E // NW

_MESH = plsc.VectorSubcoreMesh(core_axis_name="c", subcore_axis_name="s")
_F32 = jnp.float32


def _slabbed(sid, emit):
    @pl.when(sid < NS - 1)
    def _():
        emit(pl.multiple_of(sid * SLAB_A, 8), SLAB_A)

    @pl.when(sid == NS - 1)
    def _():
        emit((NS - 1) * SLAB_A, SLAB_B)


def _make_deg():
    @functools.partial(
        pl.kernel,
        out_type=[jax.ShapeDtypeStruct((N, D), _F32),
                  jax.ShapeDtypeStruct((N, D), _F32)],
        mesh=_MESH,
        scratch_types=[
            pltpu.VMEM((NCH_E, CH), jnp.int32),
            pltpu.VMEM((CH, D), _F32),
            pltpu.VMEM((CH, D), _F32),
            pltpu.VMEM_SHARED((N, D), _F32),
        ],
    )
    def degk(dstd, out0, out1, dst_v, ones_v, zeros_v, acc):
        cid = lax.axis_index("c")
        sid = lax.axis_index("s")
        w = cid * NS + sid
        pltpu.sync_copy(dstd.at[w], dst_v)
        _fill2d(ones_v, CH, 1.0)
        _fill2d(zeros_v, CH, 0.0)
        _slabbed(sid, lambda r0, n: _zero_slab(zeros_v, acc, r0, n))
        plsc.subcore_barrier()

        def body(i, _):
            pltpu.sync_copy(ones_v, acc.at[dst_v.at[i]], add=True)
            return 0
        lax.fori_loop(0, NCH_E, body, 0)
        plsc.subcore_barrier()

        @pl.when(cid == 0)
        def _():
            _slabbed(sid, lambda r0, n: pltpu.sync_copy(
                acc.at[pl.ds(r0, n)], out0.at[pl.ds(r0, n)]))

        @pl.when(cid == 1)
        def _():
            _slabbed(sid, lambda r0, n: pltpu.sync_copy(
                acc.at[pl.ds(r0, n)], out1.at[pl.ds(r0, n)]))

    return degk


def _fill2d(ref, nrows, val):
    def body(i, _):
        r = i // 8
        c = lax.rem(i, 8) * 16
        ref[r, pl.ds(c, 16)] = jnp.full((16,), val, _F32)
        return 0
    lax.fori_loop(0, nrows * 8, body, 0)


def _zero_slab(zeros_v, acc, r0, n):
    for off in range(0, n, CH):
        m = min(CH, n - off)
        pltpu.sync_copy(zeros_v.at[pl.ds(0, m)], acc.at[pl.ds(r0 + off, m)])


def _edge_phase(table, acc, src3, dst3, w, nstages,
                src_v, dst_v, rows, sems):
    for st in range(nstages):
        pltpu.sync_copy(src3.at[w * nstages + st], src_v)
        pltpu.sync_copy(dst3.at[w * nstages + st], dst_v)
        _edge_ring(table, acc, src_v, dst_v, rows, sems, NSC)


def _edge_ring(table, acc, src_v, dst_v, rows, sems, nch):
    rb = len(rows)
    for k in range(rb):
        pltpu.async_copy(table.at[src_v.at[k]], rows[k], sems[k])

    def body(p, _):
        for k in range(rb):
            i = rb * p + k
            pltpu.make_async_copy(
                table.at[src_v.at[i]], rows[k], sems[k]).wait()
            pltpu.sync_copy(rows[k], acc.at[dst_v.at[i]], add=True)

            @pl.when(i + rb < nch)
            def _():
                pltpu.async_copy(table.at[src_v.at[i + rb]], rows[k], sems[k])
        return 0
    lax.fori_loop(0, nch // rb, body, 0)
    for k in range(nch % rb):
        i = (nch // rb) * rb + k
        pltpu.make_async_copy(table.at[src_v.at[i]], rows[k], sems[k]).wait()
        pltpu.sync_copy(rows[k], acc.at[dst_v.at[i]], add=True)


def _make_spmm_e():
    @functools.partial(
        pl.kernel,
        out_type=[jax.ShapeDtypeStruct((N, D), _F32),
                  jax.ShapeDtypeStruct((N, D), _F32)],
        mesh=_MESH,
        scratch_types=[
            pltpu.VMEM((NSC, CH), jnp.int32),
            pltpu.VMEM((NSC, CH), jnp.int32),
            pltpu.VMEM((CH, D), _F32),
            pltpu.VMEM((CH, D), _F32),
            pltpu.VMEM((CH, D), _F32),
            pltpu.VMEM_SHARED((N, D), _F32),
            pltpu.SemaphoreType.DMA,
            pltpu.SemaphoreType.DMA,
            pltpu.SemaphoreType.DMA,
        ],
    )
    def spmm_e(xs, src3, dst3, zhbm, out0, out1,
               src_v, dst_v, rows0, rows1, rows2, acc, sem0, sem1, sem2):
        cid = lax.axis_index("c")
        sid = lax.axis_index("s")
        w = cid * NS + sid

        @pl.when(cid == 0)
        def _():
            _slabbed(sid, lambda r0, n: pltpu.sync_copy(
                xs.at[pl.ds(r0, n)], acc.at[pl.ds(r0, n)]))

        @pl.when(cid == 1)
        def _():
            _slabbed(sid, lambda r0, n: pltpu.sync_copy(
                zhbm.at[pl.ds(0, n)], acc.at[pl.ds(r0, n)]))

        plsc.subcore_barrier()
        _edge_phase(xs, acc, src3, dst3, w, STG_E, src_v, dst_v,
                    (rows0, rows1, rows2), (sem0, sem1, sem2))
        plsc.subcore_barrier()

        @pl.when(cid == 0)
        def _():
            _slabbed(sid, lambda r0, n: pltpu.sync_copy(
                acc.at[pl.ds(r0, n)], out0.at[pl.ds(r0, n)]))

        @pl.when(cid == 1)
        def _():
            _slabbed(sid, lambda r0, n: pltpu.sync_copy(
                acc.at[pl.ds(r0, n)], out1.at[pl.ds(r0, n)]))

    return spmm_e


def _make_spmm_f():
    @functools.partial(
        pl.kernel,
        out_type=[jax.ShapeDtypeStruct((N, D), _F32),
                  jax.ShapeDtypeStruct((N, D), _F32)],
        mesh=_MESH,
        scratch_types=[
            pltpu.VMEM((NSC, CH), jnp.int32),
            pltpu.VMEM((NSC, CH), jnp.int32),
            pltpu.VMEM((CH, D), _F32),
            pltpu.VMEM((CH, D), _F32),
            pltpu.VMEM((CH, D), _F32),
            pltpu.VMEM_SHARED((N, D), _F32),
            pltpu.SemaphoreType.DMA,
            pltpu.SemaphoreType.DMA,
            pltpu.SemaphoreType.DMA,
        ],
    )
    def spmm_f(xs0, xs1, src3, dst3, out0, out1,
               src_v, dst_v, rows0, rows1, rows2, acc, sem0, sem1, sem2):
        cid = lax.axis_index("c")
        sid = lax.axis_index("s")

        @pl.when(cid == 0)
        def _():
            _slabbed(sid, lambda r0, n: pltpu.sync_copy(
                xs0.at[pl.ds(r0, n)], acc.at[pl.ds(r0, n)]))

        @pl.when(cid == 1)
        def _():
            _slabbed(sid, lambda r0, n: pltpu.sync_copy(
                xs1.at[pl.ds(r0, n)], acc.at[pl.ds(r0, n)]))

        plsc.subcore_barrier()

        @pl.when(cid == 0)
        def _():
            _edge_phase(xs0, acc, src3, dst3, sid, STG_F, src_v, dst_v,
                        (rows0, rows1, rows2), (sem0, sem1, sem2))

        @pl.when(cid == 1)
        def _():
            _edge_phase(xs1, acc, src3, dst3, sid, STG_F, src_v, dst_v,
                        (rows0, rows1, rows2), (sem0, sem1, sem2))

        plsc.subcore_barrier()

        @pl.when(cid == 0)
        def _():
            _slabbed(sid, lambda r0, n: pltpu.sync_copy(
                acc.at[pl.ds(r0, n)], out0.at[pl.ds(r0, n)]))

        @pl.when(cid == 1)
        def _():
            _slabbed(sid, lambda r0, n: pltpu.sync_copy(
                acc.at[pl.ds(r0, n)], out1.at[pl.ds(r0, n)]))

    return spmm_f


_DEG = _make_deg()
_SPMM_E = _make_spmm_e()
_SPMM_F = _make_spmm_f()

_BM = 1000


def _scale(x, h0, h1):
    def body(x_ref, h0_ref, h1_ref, oxs, od):
        deg = (h0_ref[:, :1] + h1_ref[:, :1]).astype(_F32) + 1.0
        di = lax.rsqrt(deg)
        oxs[...] = x_ref[...] * di
        od[...] = di

    return pl.pallas_call(
        body,
        grid=(N // _BM,),
        in_specs=[pl.BlockSpec((_BM, D), lambda m: (m, 0)),
                  pl.BlockSpec((_BM, D), lambda m: (m, 0)),
                  pl.BlockSpec((_BM, D), lambda m: (m, 0))],
        out_specs=[pl.BlockSpec((_BM, D), lambda m: (m, 0)),
                   pl.BlockSpec((_BM, 1), lambda m: (m, 0))],
        out_shape=[jax.ShapeDtypeStruct((N, D), _F32),
                   jax.ShapeDtypeStruct((N, 1), _F32)],
    )(x, h0, h1)


def _layer1(a0, a1, dinv, w1t, b1r):
    def body(a0_ref, a1_ref, dv, w, b, o0, o1):
        di = dv[...]
        agg = (a0_ref[...] + a1_ref[...]) * di
        h = jnp.dot(agg, w[...], preferred_element_type=_F32) + b[...]
        xs2 = jnp.maximum(h, 0.0) * di
        o0[...] = xs2[:, :D]
        o1[...] = xs2[:, D:]

    return pl.pallas_call(
        body,
        grid=(N // _BM,),
        in_specs=[pl.BlockSpec((_BM, D), lambda m: (m, 0)),
                  pl.BlockSpec((_BM, D), lambda m: (m, 0)),
                  pl.BlockSpec((_BM, 1), lambda m: (m, 0)),
                  pl.BlockSpec((D, 256), lambda m: (0, 0)),
                  pl.BlockSpec((1, 256), lambda m: (0, 0))],
        out_specs=[pl.BlockSpec((_BM, D), lambda m: (m, 0)),
                   pl.BlockSpec((_BM, D), lambda m: (m, 0))],
        out_shape=[jax.ShapeDtypeStruct((N, D), _F32),
                   jax.ShapeDtypeStruct((N, D), _F32)],
    )(a0, a1, dinv, w1t, b1r)


def _layer2(g0, g1, dinv, w2t, b2r, w3t):
    def body(g0_ref, g1_ref, dv, w2, b2, w3, o):
        di = dv[...]
        agg = jnp.concatenate([g0_ref[...], g1_ref[...]], axis=1) * di
        h2 = jnp.maximum(
            jnp.dot(agg, w2[...], preferred_element_type=_F32) + b2[...], 0.0)
        y = jnp.dot(h2, w3[...], preferred_element_type=_F32)
        o[...] = y * di

    return pl.pallas_call(
        body,
        grid=(N // _BM,),
        in_specs=[pl.BlockSpec((_BM, D), lambda m: (m, 0)),
                  pl.BlockSpec((_BM, D), lambda m: (m, 0)),
                  pl.BlockSpec((_BM, 1), lambda m: (m, 0)),
                  pl.BlockSpec((256, 256), lambda m: (0, 0)),
                  pl.BlockSpec((1, 256), lambda m: (0, 0)),
                  pl.BlockSpec((256, D), lambda m: (0, 0))],
        out_specs=pl.BlockSpec((_BM, D), lambda m: (m, 0)),
        out_shape=jax.ShapeDtypeStruct((N, D), _F32),
    )(g0, g1, dinv, w2t, b2r, w3t)


def _layer3(c0, c1, dinv, b3r):
    def body(c0_ref, c1_ref, dv, b, o):
        di = dv[...]
        agg = (c0_ref[...] + c1_ref[...]) * di
        o[...] = jnp.maximum(agg + b[...], 0.0)

    return pl.pallas_call(
        body,
        grid=(N // _BM,),
        in_specs=[pl.BlockSpec((_BM, D), lambda m: (m, 0)),
                  pl.BlockSpec((_BM, D), lambda m: (m, 0)),
                  pl.BlockSpec((_BM, 1), lambda m: (m, 0)),
                  pl.BlockSpec((1, D), lambda m: (0, 0))],
        out_specs=pl.BlockSpec((_BM, D), lambda m: (m, 0)),
        out_shape=jax.ShapeDtypeStruct((N, D), _F32),
    )(c0, c1, dinv, b3r)


def _mlp(hr, w1t, b1r, w2t, b2r, w3row, b3s):
    def body(h, w1, b1, w2, b2, w3, b3, o):
        z = jnp.dot(h[...], w1[...], preferred_element_type=_F32) + b1[...]
        z = jnp.where(z > 0, z, 0.1 * z)
        z = jnp.dot(z, w2[...], preferred_element_type=_F32) + b2[...]
        z = jnp.where(z > 0, z, 0.1 * z)
        t = jnp.sum(z * w3[...], axis=1, keepdims=True) + b3[...]
        o[...] = 1.0 / (1.0 + jnp.exp(-t))

    m = hr.shape[0]
    return pl.pallas_call(
        body,
        out_shape=jax.ShapeDtypeStruct((m, 1), _F32),
    )(hr, w1t, b1r, w2t, b2r, w3row, b3s)


def kernel(x, edge_index, W1, b1, W2, b2, W3, b3,
           L1W, L1b, L2W, L2b, L3W, L3b):
    src = edge_index[0]
    dst = edge_index[1]
    src3 = src.reshape(E // (NSC * CH), NSC, CH)
    dst3 = dst.reshape(E // (NSC * CH), NSC, CH)
    dstd = dst.reshape(NW, NCH_E, CH)
    zhbm = jnp.zeros((SLAB_B, D), _F32)

    h0, h1 = _DEG(dstd)
    xs, dinv = _scale(x, h0, h1)
    a0, a1 = _SPMM_E(xs, src3, dst3, zhbm)
    x2a, x2b = _layer1(a0, a1, dinv, W1.T, b1.reshape(1, -1))
    g0, g1 = _SPMM_F(x2a, x2b, src3, dst3)
    xs3 = _layer2(g0, g1, dinv, W2.T, b2.reshape(1, -1), W3.T)
    c0, c1 = _SPMM_E(xs3, src3, dst3, zhbm)
    h3 = _layer3(c0, c1, dinv, b3.reshape(1, -1))
    hr = h3.reshape(N // 40, 40 * D)
    return _mlp(hr, L1W.T, L1b.reshape(1, -1), L2W.T, L2b.reshape(1, -1),
                L3W.reshape(1, -1), L3b.reshape(1, 1))
